# baseline (speedup 1.0000x reference)
import functools

import jax
import jax.numpy as jnp
from jax import lax
from jax.experimental import pallas as pl
from jax.experimental.pallas import tpu as pltpu
from jax.experimental.pallas import tpu_sc as plsc

_NS, _N = 16384, 1024
_L = 16
_NC, _NSUB = 2, 16
_NW = _NC * _NSUB
_RPW = _NS // _NW
_CH = 16
_NCHUNK = _RPW // _CH
_NVEC = _N // _L
_UNROLL = 4


def _prep_body(phi_t_ref, out_ref):
    lp = jnp.log(phi_t_ref[...])
    out_ref[0:1, :] = 0.5 * (lp[1:2, :] - lp[0:1, :])
    c0 = 0.5 * jnp.sum(lp)
    out_ref[1:2, :] = jnp.broadcast_to(c0, (1, _N))


_prep = pl.pallas_call(
    _prep_body,
    out_shape=jax.ShapeDtypeStruct((2, _N), jnp.float32),
)


def _sc_body(x_hbm, prep_hbm, out_hbm, xbuf, dh_v, c0_v, out_v, tr_v,
             sem0, sem1):
    wid = lax.axis_index("s") * _NC + lax.axis_index("c")
    base = wid * _RPW

    pltpu.sync_copy(prep_hbm.at[0], dh_v)
    pltpu.sync_copy(prep_hbm.at[1], c0_v)
    sems = (sem0, sem1)

    def start(g, b):
        pltpu.async_copy(
            x_hbm.at[pl.ds(base + g * _CH, _CH), :], xbuf.at[b], sems[b])

    def wait(b):
        pltpu.make_async_copy(
            x_hbm.at[pl.ds(base, _CH), :], xbuf.at[b], sems[b]).wait()

    start(0, 0)
    start(1, 1)

    def compute_chunk(g, b):
        wait(b)

        def inner(t, accs):
            for u in range(_UNROLL):
                j = t * _UNROLL + u
                dv = dh_v[pl.ds(j * _L, _L)]
                accs = tuple(
                    accs[r]
                    + dv * xbuf[b, r, pl.ds(j * _L, _L)].astype(jnp.float32)
                    for r in range(_CH))
            return accs

        accs = lax.fori_loop(
            0, _NVEC // _UNROLL, inner,
            tuple(jnp.zeros((_L,), jnp.float32) for _ in range(_CH)))

        @pl.when(g + 2 < _NCHUNK)
        def _():
            start(g + 2, b)

        lane = lax.iota(jnp.int32, _L)
        for grp in range(_CH // _L):
            for r in range(_L):
                tr_v[r, :] = accs[grp * _L + r]
            res = c0_v[pl.ds(0, _L)]
            for k in range(_L):
                res = res + plsc.load_gather(
                    tr_v, [lane, jnp.full((_L,), k, jnp.int32)])
            out_v[pl.ds(g * _CH + grp * _L, _L)] = res

    def outer(t, carry):
        compute_chunk(2 * t, 0)
        compute_chunk(2 * t + 1, 1)
        return carry

    lax.fori_loop(0, _NCHUNK // 2, outer, 0)

    pltpu.sync_copy(out_v, out_hbm.at[pl.ds(base, _RPW)])


@functools.cache
def _sc():
    return functools.partial(
        pl.kernel,
        out_type=jax.ShapeDtypeStruct((_NS,), jnp.float32),
        mesh=plsc.VectorSubcoreMesh(
            core_axis_name="c", subcore_axis_name="s",
            num_cores=_NC, num_subcores=_NSUB),
        scratch_types=[
            pltpu.VMEM((2, _CH, _N), jnp.int32),
            pltpu.VMEM((_N,), jnp.float32),
            pltpu.VMEM((_N,), jnp.float32),
            pltpu.VMEM((_RPW,), jnp.float32),
            pltpu.VMEM((_L, _L), jnp.float32),
            pltpu.SemaphoreType.DMA,
            pltpu.SemaphoreType.DMA,
        ],
        compiler_params=pltpu.CompilerParams(needs_layout_passes=False),
    )(_sc_body)


def kernel(x, phi):
    prep = _prep(phi.T)
    return _sc()(x, prep)


# speedup vs baseline: 1.1081x; 1.1081x over previous
import functools

import jax
import jax.numpy as jnp
from jax import lax
from jax.experimental import pallas as pl
from jax.experimental.pallas import tpu as pltpu
from jax.experimental.pallas import tpu_sc as plsc

_NS, _N = 16384, 1024
_L = 16
_NC, _NSUB = 2, 16
_NW = _NC * _NSUB
_RPW = _NS // _NW
_CH = 16
_NCHUNK = _RPW // _CH
_NVEC = _N // _L

_LN2 = 0.6931471805599453
_SQRT2 = 1.4142135623730951


def _log16(v):
    bits = plsc.bitcast(v, jnp.int32)
    e = ((bits >> 23) & 0xFF) - 127
    m = plsc.bitcast((bits & 0x007FFFFF) | 0x3F800000, jnp.float32)
    big = m >= _SQRT2
    e = e + big.astype(jnp.int32)
    m = jnp.where(big, m * 0.5, m)
    t = m - 1.0
    s = t / (t + 2.0)
    s2 = s * s
    p = 2.0 + s2 * (0.66666667 + s2 * (0.4 + s2 * 0.28571429))
    return e.astype(jnp.float32) * _LN2 + s * p


def _sc_body(x_hbm, phi_hbm, out_hbm, xbuf, phi_v, dh_v, out_v, tr_v,
             sem0, sem1):
    wid = lax.axis_index("s") * _NC + lax.axis_index("c")
    base = wid * _RPW
    sems = (sem0, sem1)

    def start(g, b):
        pltpu.async_copy(
            x_hbm.at[pl.ds(base + g * _CH, _CH), :], xbuf.at[b], sems[b])

    def wait(b):
        pltpu.make_async_copy(
            x_hbm.at[pl.ds(base, _CH), :], xbuf.at[b], sems[b]).wait()

    start(0, 0)
    start(1, 1)

    pltpu.sync_copy(phi_hbm, phi_v)

    lane = lax.iota(jnp.int32, _L)

    def prep(j, csum):
        idx2 = (j * _L + lane) * 2
        la = _log16(plsc.load_gather(phi_v, [idx2]))
        lb = _log16(plsc.load_gather(phi_v, [idx2 + 1]))
        dh_v[pl.ds(j * _L, _L)] = 0.5 * (lb - la)
        return csum + 0.5 * (la + lb)

    csum = lax.fori_loop(0, _NVEC, prep, jnp.zeros((_L,), jnp.float32))

    for r in range(_L):
        tr_v[r, :] = csum
    c0vec = jnp.zeros((_L,), jnp.float32)
    for k in range(_L):
        c0vec = c0vec + plsc.load_gather(
            tr_v, [lane, jnp.full((_L,), k, jnp.int32)])

    def compute_chunk(g, b):
        wait(b)

        def inner(j, accs):
            dv = dh_v[pl.ds(j * _L, _L)]
            return tuple(
                accs[r] + dv * xbuf[b, r, pl.ds(j * _L, _L)].astype(jnp.float32)
                for r in range(_CH))

        accs = lax.fori_loop(
            0, _NVEC, inner,
            tuple(jnp.zeros((_L,), jnp.float32) for _ in range(_CH)))

        @pl.when(g + 2 < _NCHUNK)
        def _():
            start(g + 2, b)

        for r in range(_L):
            tr_v[r, :] = accs[r]
        res = c0vec
        for k in range(_L):
            res = res + plsc.load_gather(
                tr_v, [lane, jnp.full((_L,), k, jnp.int32)])
        out_v[pl.ds(g * _CH, _L)] = res

    def outer(t, carry):
        compute_chunk(2 * t, 0)
        compute_chunk(2 * t + 1, 1)
        return carry

    lax.fori_loop(0, _NCHUNK // 2, outer, 0)

    pltpu.sync_copy(out_v, out_hbm.at[pl.ds(base, _RPW)])


@functools.cache
def _sc():
    return functools.partial(
        pl.kernel,
        out_type=jax.ShapeDtypeStruct((_NS,), jnp.float32),
        mesh=plsc.VectorSubcoreMesh(
            core_axis_name="c", subcore_axis_name="s",
            num_cores=_NC, num_subcores=_NSUB),
        scratch_types=[
            pltpu.VMEM((2, _CH, _N), jnp.int32),
            pltpu.VMEM((2 * _N,), jnp.float32),
            pltpu.VMEM((_N,), jnp.float32),
            pltpu.VMEM((_RPW,), jnp.float32),
            pltpu.VMEM((_L, _L), jnp.float32),
            pltpu.SemaphoreType.DMA,
            pltpu.SemaphoreType.DMA,
        ],
        compiler_params=pltpu.CompilerParams(needs_layout_passes=False),
    )(_sc_body)


def kernel(x, phi):
    return _sc()(x, phi.reshape(-1))
